# baseline (speedup 1.0000x reference)
import functools

import jax
import jax.numpy as jnp
from jax import lax
from jax.experimental import pallas as pl
from jax.experimental.pallas import tpu as pltpu
from jax.experimental.pallas import tpu_sc as plsc

B, H, S = 2, 12, 2048
L = 16
NCHUNK = S // L
NBIN = 256
NBVREG = NBIN // L
NEG_THRESH = -10000000000.0
MIN32 = -(2**31)
LOW31 = 0x7FFFFFFF


def _monotone_key(x):
    b = lax.bitcast_convert_type(x, jnp.int32)
    return b ^ ((b >> 31) & LOW31)


def _worker_body(b, att_hbm, out_hbm, rows_v, keys_v, hist_v, hist2d_v,
                 cand1_v, cand2_v, eqc_v, sfx_v, out_v, sem):
    ones = jnp.ones((L,), jnp.int32)
    zeros = jnp.zeros((L,), jnp.int32)
    big = jnp.full((L,), LOW31, jnp.int32)
    iota = lax.broadcasted_iota(jnp.int32, (L,), 0)
    lane_base = iota * NBIN

    with jax.named_scope("ph_dma"):
        cp = pltpu.async_copy(att_hbm.at[b, :, 0, :], rows_v, sem)
        for i in range(NBIN):
            hist2d_v[pl.ds(i * L, L)] = zeros
        cp.wait()

    def score_body(c, cnt):
        base = c * L
        r = [rows_v[h, pl.ds(base, L)] for h in range(H)]
        s01, s23, s45 = r[0] + r[1], r[2] + r[3], r[4] + r[5]
        s67, s89, sab = r[6] + r[7], r[8] + r[9], r[10] + r[11]
        acc = ((s01 + s23) + (s45 + s67)) + (s89 + sab)
        key = _monotone_key(acc)
        keys_v[pl.ds(base, L)] = key
        ub = key ^ MIN32
        byte3 = lax.shift_right_logical(ub, 24)
        plsc.addupdate_scatter(hist2d_v, [lane_base + byte3], ones)
        return cnt + jnp.where(r[0] > NEG_THRESH, ones, zeros)

    with jax.named_scope("ph_score"):
        cntv = lax.fori_loop(0, NCHUNK, score_body, jnp.zeros((L,), jnp.int32),
                             unroll=8)
    num_tokens = jnp.sum(cntv)
    num_topk = (num_tokens + 9) // 10

    def red_body(g, carry):
        accs = [hist2d_v[pl.ds(l * NBIN + g * L, L)] for l in range(L)]
        t = [accs[2 * i] + accs[2 * i + 1] for i in range(8)]
        t = [t[2 * i] + t[2 * i + 1] for i in range(4)]
        t = [t[2 * i] + t[2 * i + 1] for i in range(2)]
        hist_v[pl.ds(g * L, L)] = t[0] + t[1]
        return carry

    with jax.named_scope("ph_red"):
        lax.fori_loop(0, NBVREG, red_body, 0, unroll=4)

    def pick_digit(remaining):
        def scan_body(i, carry):
            acc, found, digit, greater, bincnt = carry
            h = hist_v[pl.ds(240 - 16 * i, L)]
            rh = lax.rev(h, (0,))
            rcs = plsc.cumsum(rh)
            g_cs = acc + rcs
            ge = g_cs >= remaining
            pc = plsc.all_reduce_population_count(ge)
            hit_cnt = pc[0]
            m_local = 16 - hit_cnt
            loc_digit = 255 - 16 * i - m_local
            loc_cs = jnp.min(jnp.where(ge, g_cs, big))
            loc_greater = jnp.min(jnp.where(ge, g_cs - rh, big))
            loc_binc = loc_cs - loc_greater
            upd = (hit_cnt > 0) & (found == 0)
            digit = jnp.where(upd, loc_digit, digit)
            greater = jnp.where(upd, loc_greater, greater)
            bincnt = jnp.where(upd, loc_binc, bincnt)
            found = found | jnp.where(hit_cnt > 0, 1, 0)
            acc = g_cs[15]
            return acc, found, digit, greater, bincnt

        z = jnp.int32(0)
        _, _, digit, greater, bincnt = lax.fori_loop(
            0, NBVREG, scan_body, (z, z, z, z, z), unroll=4)
        return digit, greater, bincnt

    with jax.named_scope("ph_pick0"):
        digit, greater, bincnt = pick_digit(num_topk)
    prefix = digit
    remaining = num_topk - greater

    for i in range(NBVREG):
        hist_v[pl.ds(i * L, L)] = zeros

    def hist2_body(c, off, prefix=prefix):
        k = keys_v[pl.ds(c * L, L)]
        ub = k ^ MIN32
        byte = lax.shift_right_logical(ub, 16) & 0xFF
        elig = lax.shift_right_logical(ub, 24) == prefix
        plsc.addupdate_scatter(hist_v, [byte], ones, mask=elig)
        plsc.store_compressed(cand1_v.at[pl.ds(off, L)], k, mask=elig)
        pc = plsc.all_reduce_population_count(elig)
        return off + pc[0]

    with jax.named_scope("ph_hist2"):
        n1 = lax.fori_loop(0, NCHUNK, hist2_body, jnp.int32(0), unroll=8)
    with jax.named_scope("ph_pick"):
        digit, greater, bincnt = pick_digit(remaining)
    prefix = (prefix << 8) | digit
    remaining = remaining - greater

    for i in range(NBVREG):
        hist_v[pl.ds(i * L, L)] = zeros

    def hist1_body(c, off, prefix=prefix):
        base = c * L
        k = cand1_v[pl.ds(base, L)]
        ub = k ^ MIN32
        byte = lax.shift_right_logical(ub, 8) & 0xFF
        valid = (base + iota) < n1
        elig = (lax.shift_right_logical(ub, 16) == prefix) & valid
        plsc.addupdate_scatter(hist_v, [byte], ones, mask=elig)
        plsc.store_compressed(cand2_v.at[pl.ds(off, L)], k, mask=elig)
        pc = plsc.all_reduce_population_count(elig)
        return off + pc[0]

    nv1 = (n1 + L - 1) // L
    with jax.named_scope("ph_hist1"):
        n2 = lax.fori_loop(0, nv1, hist1_body, jnp.int32(0))
    with jax.named_scope("ph_pick"):
        digit, greater, bincnt = pick_digit(remaining)
    prefix = (prefix << 8) | digit
    remaining = remaining - greater

    for i in range(NBVREG):
        hist_v[pl.ds(i * L, L)] = zeros

    def hist0_body(c, carry, prefix=prefix):
        base = c * L
        k = cand2_v[pl.ds(base, L)]
        ub = k ^ MIN32
        byte = ub & 0xFF
        valid = (base + iota) < n2
        elig = (lax.shift_right_logical(ub, 8) == prefix) & valid
        plsc.addupdate_scatter(hist_v, [byte], ones, mask=elig)
        return carry

    nv2 = (n2 + L - 1) // L
    with jax.named_scope("ph_hist0"):
        lax.fori_loop(0, nv2, hist0_body, 0)
    with jax.named_scope("ph_pick"):
        digit, greater, bincnt = pick_digit(remaining)
    prefix = (prefix << 8) | digit
    remaining = remaining - greater

    t_key = prefix ^ MIN32
    m_eq = remaining
    total_eq = bincnt

    kvec = jnp.broadcast_to(num_topk, (L,)).astype(jnp.float32)
    invv = jnp.full((L,), 1.0, jnp.float32) / kvec
    zerofv = jnp.zeros((L,), jnp.float32)

    @pl.when(m_eq == total_eq)
    def _no_tie():
        def out_body(c, carry):
            k = keys_v[pl.ds(c * L, L)]
            out_v[pl.ds(c * L, L)] = jnp.where(k >= t_key, invv, zerofv)
            return carry

        with jax.named_scope("ph_out"):
            lax.fori_loop(0, NCHUNK, out_body, 0, unroll=8)

    @pl.when(m_eq != total_eq)
    def _tie():
        def eqcnt_body(c, tot):
            k = keys_v[pl.ds(c * L, L)]
            eq = k == t_key
            pc = plsc.all_reduce_population_count(eq)
            e = pc[0]
            eqc_v[c] = e
            return tot + e

        tot = lax.fori_loop(0, NCHUNK, eqcnt_body, jnp.int32(0), unroll=4)

        def sfx_body(c, run):
            e = eqc_v[c]
            sfx_v[c] = tot - run - e
            return run + e

        lax.fori_loop(0, NCHUNK, sfx_body, jnp.int32(0), unroll=4)

        def out_body(c, carry):
            k = keys_v[pl.ds(c * L, L)]
            eq = k == t_key
            eqi = jnp.where(eq, ones, zeros)
            at_or_after = lax.rev(plsc.cumsum(lax.rev(eqi, (0,))), (0,))
            eq_after = at_or_after - eqi + sfx_v[c]
            sel = (k > t_key) | (eq & (eq_after < m_eq))
            out_v[pl.ds(c * L, L)] = jnp.where(sel, invv, zerofv)
            return carry

        lax.fori_loop(0, NCHUNK, out_body, 0, unroll=4)

    pltpu.sync_copy(out_v, out_hbm.at[b])


def _make_sc_kernel(interpret=False):
    mesh = plsc.VectorSubcoreMesh(core_axis_name="c", subcore_axis_name="s",
                                  num_cores=2, num_subcores=16)

    @functools.partial(
        pl.kernel,
        out_type=jax.ShapeDtypeStruct((B, S), jnp.float32),
        mesh=mesh,
        scratch_types=[
            pltpu.VMEM((H, S), jnp.float32),
            pltpu.VMEM((S,), jnp.int32),
            pltpu.VMEM((NBIN,), jnp.int32),
            pltpu.VMEM((NBIN * L,), jnp.int32),
            pltpu.VMEM((S + L,), jnp.int32),
            pltpu.VMEM((S + L,), jnp.int32),
            pltpu.SMEM((NCHUNK,), jnp.int32),
            pltpu.SMEM((NCHUNK,), jnp.int32),
            pltpu.VMEM((S,), jnp.float32),
            pltpu.SemaphoreType.DMA,
        ],
        compiler_params=pltpu.CompilerParams(needs_layout_passes=False),
        interpret=interpret,
    )
    def sc_topk(att_hbm, out_hbm, rows_v, keys_v, hist_v, hist2d_v, cand1_v,
                cand2_v, eqc_v, sfx_v, out_v, sem):
        wid = lax.axis_index("s") * 2 + lax.axis_index("c")

        @pl.when(wid < B)
        def _():
            _worker_body(wid, att_hbm, out_hbm, rows_v, keys_v, hist_v,
                         hist2d_v, cand1_v, cand2_v, eqc_v, sfx_v, out_v, sem)

    return sc_topk


_sc_topk = _make_sc_kernel()


def kernel(attention):
    return (_sc_topk(attention), None)


# speedup vs baseline: 1.1565x; 1.1565x over previous
import functools

import jax
import jax.numpy as jnp
from jax import lax
from jax.experimental import pallas as pl
from jax.experimental.pallas import tpu as pltpu
from jax.experimental.pallas import tpu_sc as plsc

B, H, S = 2, 12, 2048
L = 16
NCHUNK = S // L
NBIN = 256
NBVREG = NBIN // L
NEG_THRESH = -10000000000.0
MIN32 = -(2**31)
LOW31 = 0x7FFFFFFF


def _monotone_key(x):
    b = lax.bitcast_convert_type(x, jnp.int32)
    return b ^ ((b >> 31) & LOW31)


def _worker_body(b, att_hbm, out_hbm, rows_v, keys_v, hist_v, cand1_v, cand2_v,
                 eqc_v, sfx_v, out_v, sem):
    ones = jnp.ones((L,), jnp.int32)
    zeros = jnp.zeros((L,), jnp.int32)
    big = jnp.full((L,), LOW31, jnp.int32)
    iota = lax.broadcasted_iota(jnp.int32, (L,), 0)

    with jax.named_scope("ph_dma"):
        pltpu.sync_copy(att_hbm.at[b, :, 0, :], rows_v)

    for i in range(NBVREG):
        hist_v[pl.ds(i * L, L)] = zeros

    def score_body(c, cnt):
        base = c * L
        r = [rows_v[h, pl.ds(base, L)] for h in range(H)]
        s01, s23, s45 = r[0] + r[1], r[2] + r[3], r[4] + r[5]
        s67, s89, sab = r[6] + r[7], r[8] + r[9], r[10] + r[11]
        acc = ((s01 + s23) + (s45 + s67)) + (s89 + sab)
        key = _monotone_key(acc)
        keys_v[pl.ds(base, L)] = key
        ub = key ^ MIN32
        byte3 = lax.shift_right_logical(ub, 24)
        plsc.addupdate_scatter(hist_v, [byte3], ones)
        return cnt + jnp.where(r[0] > NEG_THRESH, ones, zeros)

    with jax.named_scope("ph_score"):
        @plsc.parallel_loop(0, NCHUNK, unroll=8,
                            carry=jnp.zeros((L,), jnp.int32))
        def cntv(c, cnt):
            return score_body(c, cnt)
    num_tokens = jnp.sum(cntv)
    num_topk = (num_tokens + 9) // 10

    def pick_digit(remaining):
        def scan_body(i, carry):
            acc, found, digit, greater, bincnt = carry
            h = hist_v[pl.ds(240 - 16 * i, L)]
            rh = lax.rev(h, (0,))
            rcs = plsc.cumsum(rh)
            g_cs = acc + rcs
            ge = g_cs >= remaining
            pc = plsc.all_reduce_population_count(ge)
            hit_cnt = pc[0]
            m_local = 16 - hit_cnt
            loc_digit = 255 - 16 * i - m_local
            loc_cs = jnp.min(jnp.where(ge, g_cs, big))
            loc_greater = jnp.min(jnp.where(ge, g_cs - rh, big))
            loc_binc = loc_cs - loc_greater
            upd = (hit_cnt > 0) & (found == 0)
            digit = jnp.where(upd, loc_digit, digit)
            greater = jnp.where(upd, loc_greater, greater)
            bincnt = jnp.where(upd, loc_binc, bincnt)
            found = found | jnp.where(hit_cnt > 0, 1, 0)
            acc = g_cs[15]
            return acc, found, digit, greater, bincnt

        z = jnp.int32(0)
        _, _, digit, greater, bincnt = lax.fori_loop(
            0, NBVREG, scan_body, (z, z, z, z, z), unroll=4)
        return digit, greater, bincnt

    with jax.named_scope("ph_pick0"):
        digit, greater, bincnt = pick_digit(num_topk)
    prefix = digit
    remaining = num_topk - greater

    for i in range(NBVREG):
        hist_v[pl.ds(i * L, L)] = zeros

    def hist2_body(c, off, prefix=prefix):
        k = keys_v[pl.ds(c * L, L)]
        ub = k ^ MIN32
        byte = lax.shift_right_logical(ub, 16) & 0xFF
        elig = lax.shift_right_logical(ub, 24) == prefix
        plsc.addupdate_scatter(hist_v, [byte], ones, mask=elig)
        plsc.store_compressed(cand1_v.at[pl.ds(off, L)], k, mask=elig)
        pc = plsc.all_reduce_population_count(elig)
        return off + pc[0]

    with jax.named_scope("ph_hist2"):
        @plsc.parallel_loop(0, NCHUNK, unroll=8, carry=jnp.int32(0))
        def n1(c, off):
            return hist2_body(c, off)
    with jax.named_scope("ph_pick"):
        digit, greater, bincnt = pick_digit(remaining)
    prefix = (prefix << 8) | digit
    remaining = remaining - greater

    for i in range(NBVREG):
        hist_v[pl.ds(i * L, L)] = zeros

    def hist1_body(c, off, prefix=prefix):
        base = c * L
        k = cand1_v[pl.ds(base, L)]
        ub = k ^ MIN32
        byte = lax.shift_right_logical(ub, 8) & 0xFF
        valid = (base + iota) < n1
        elig = (lax.shift_right_logical(ub, 16) == prefix) & valid
        plsc.addupdate_scatter(hist_v, [byte], ones, mask=elig)
        plsc.store_compressed(cand2_v.at[pl.ds(off, L)], k, mask=elig)
        pc = plsc.all_reduce_population_count(elig)
        return off + pc[0]

    nv1 = (n1 + L - 1) // L
    with jax.named_scope("ph_hist1"):
        n2 = lax.fori_loop(0, nv1, hist1_body, jnp.int32(0))
    with jax.named_scope("ph_pick"):
        digit, greater, bincnt = pick_digit(remaining)
    prefix = (prefix << 8) | digit
    remaining = remaining - greater

    for i in range(NBVREG):
        hist_v[pl.ds(i * L, L)] = zeros

    def hist0_body(c, carry, prefix=prefix):
        base = c * L
        k = cand2_v[pl.ds(base, L)]
        ub = k ^ MIN32
        byte = ub & 0xFF
        valid = (base + iota) < n2
        elig = (lax.shift_right_logical(ub, 8) == prefix) & valid
        plsc.addupdate_scatter(hist_v, [byte], ones, mask=elig)
        return carry

    nv2 = (n2 + L - 1) // L
    with jax.named_scope("ph_hist0"):
        lax.fori_loop(0, nv2, hist0_body, 0)
    with jax.named_scope("ph_pick"):
        digit, greater, bincnt = pick_digit(remaining)
    prefix = (prefix << 8) | digit
    remaining = remaining - greater

    t_key = prefix ^ MIN32
    m_eq = remaining
    total_eq = bincnt

    kvec = jnp.broadcast_to(num_topk, (L,)).astype(jnp.float32)
    invv = jnp.full((L,), 1.0, jnp.float32) / kvec
    zerofv = jnp.zeros((L,), jnp.float32)

    @pl.when(m_eq == total_eq)
    def _no_tie():
        def out_body(c, carry):
            k = keys_v[pl.ds(c * L, L)]
            out_v[pl.ds(c * L, L)] = jnp.where(k >= t_key, invv, zerofv)
            return carry

        with jax.named_scope("ph_out"):
            @plsc.parallel_loop(0, NCHUNK, unroll=8)
            def _outl(c):
                out_body(c, 0)

    @pl.when(m_eq != total_eq)
    def _tie():
        def eqcnt_body(c, tot):
            k = keys_v[pl.ds(c * L, L)]
            eq = k == t_key
            pc = plsc.all_reduce_population_count(eq)
            e = pc[0]
            eqc_v[c] = e
            return tot + e

        tot = lax.fori_loop(0, NCHUNK, eqcnt_body, jnp.int32(0), unroll=4)

        def sfx_body(c, run):
            e = eqc_v[c]
            sfx_v[c] = tot - run - e
            return run + e

        lax.fori_loop(0, NCHUNK, sfx_body, jnp.int32(0), unroll=4)

        def out_body(c, carry):
            k = keys_v[pl.ds(c * L, L)]
            eq = k == t_key
            eqi = jnp.where(eq, ones, zeros)
            at_or_after = lax.rev(plsc.cumsum(lax.rev(eqi, (0,))), (0,))
            eq_after = at_or_after - eqi + sfx_v[c]
            sel = (k > t_key) | (eq & (eq_after < m_eq))
            out_v[pl.ds(c * L, L)] = jnp.where(sel, invv, zerofv)
            return carry

        lax.fori_loop(0, NCHUNK, out_body, 0, unroll=4)

    pltpu.sync_copy(out_v, out_hbm.at[b])


def _make_sc_kernel(interpret=False):
    mesh = plsc.VectorSubcoreMesh(core_axis_name="c", subcore_axis_name="s",
                                  num_cores=2, num_subcores=16)

    @functools.partial(
        pl.kernel,
        out_type=jax.ShapeDtypeStruct((B, S), jnp.float32),
        mesh=mesh,
        scratch_types=[
            pltpu.VMEM((H, S), jnp.float32),
            pltpu.VMEM((S,), jnp.int32),
            pltpu.VMEM((NBIN,), jnp.int32),
            pltpu.VMEM((S + L,), jnp.int32),
            pltpu.VMEM((S + L,), jnp.int32),
            pltpu.SMEM((NCHUNK,), jnp.int32),
            pltpu.SMEM((NCHUNK,), jnp.int32),
            pltpu.VMEM((S,), jnp.float32),
            pltpu.SemaphoreType.DMA,
        ],
        compiler_params=pltpu.CompilerParams(needs_layout_passes=False),
        interpret=interpret,
    )
    def sc_topk(att_hbm, out_hbm, rows_v, keys_v, hist_v, cand1_v, cand2_v,
                eqc_v, sfx_v, out_v, sem):
        wid = lax.axis_index("s") * 2 + lax.axis_index("c")

        @pl.when(wid < B)
        def _():
            _worker_body(wid, att_hbm, out_hbm, rows_v, keys_v, hist_v,
                         cand1_v, cand2_v, eqc_v, sfx_v, out_v, sem)

    return sc_topk


_sc_topk = _make_sc_kernel()


def kernel(attention):
    return (_sc_topk(attention), None)


# speedup vs baseline: 1.1726x; 1.0140x over previous
import functools

import jax
import jax.numpy as jnp
from jax import lax
from jax.experimental import pallas as pl
from jax.experimental.pallas import tpu as pltpu
from jax.experimental.pallas import tpu_sc as plsc

B, H, S = 2, 12, 2048
L = 16
NCHUNK = S // L
NBIN = 256
NBVREG = NBIN // L
NEG_THRESH = -10000000000.0
MIN32 = -(2**31)
LOW31 = 0x7FFFFFFF


def _monotone_key(x):
    b = lax.bitcast_convert_type(x, jnp.int32)
    return b ^ ((b >> 31) & LOW31)


def _worker_body(b, att_hbm, out_hbm, rows_v, keys_v, hist_v, cand1_v, cand2_v,
                 eqc_v, sfx_v, out_v, psum_v, sem, sem2):
    ones = jnp.ones((L,), jnp.int32)
    zeros = jnp.zeros((L,), jnp.int32)
    big = jnp.full((L,), LOW31, jnp.int32)
    iota = lax.broadcasted_iota(jnp.int32, (L,), 0)

    with jax.named_scope("ph_dma"):
        cp1 = pltpu.async_copy(att_hbm.at[b, pl.ds(0, 8), 0, :],
                               rows_v.at[pl.ds(0, 8), :], sem)
        cp2 = pltpu.async_copy(att_hbm.at[b, pl.ds(8, 4), 0, :],
                               rows_v.at[pl.ds(8, 4), :], sem2)
        for i in range(NBVREG):
            hist_v[pl.ds(i * L, L)] = zeros
        cp1.wait()

    def psum_body(c, cnt):
        base = c * L
        r = [rows_v[h, pl.ds(base, L)] for h in range(8)]
        psum_v[pl.ds(base, L)] = (((r[0] + r[1]) + (r[2] + r[3]))
                                  + ((r[4] + r[5]) + (r[6] + r[7])))
        return cnt + jnp.where(r[0] > NEG_THRESH, ones, zeros)

    with jax.named_scope("ph_psum"):
        @plsc.parallel_loop(0, NCHUNK, unroll=8,
                            carry=jnp.zeros((L,), jnp.int32))
        def cntv0(c, cnt):
            return psum_body(c, cnt)

    with jax.named_scope("ph_dma2"):
        cp2.wait()

    def score_body(c, carry):
        base = c * L
        r = [rows_v[h, pl.ds(base, L)] for h in range(8, H)]
        hi = (r[0] + r[1]) + (r[2] + r[3])
        acc = psum_v[pl.ds(base, L)] + hi
        key = _monotone_key(acc)
        keys_v[pl.ds(base, L)] = key
        ub = key ^ MIN32
        byte3 = lax.shift_right_logical(ub, 24)
        plsc.addupdate_scatter(hist_v, [byte3], ones)
        return carry

    with jax.named_scope("ph_score"):
        @plsc.parallel_loop(0, NCHUNK, unroll=8)
        def _scorel(c):
            score_body(c, 0)
    num_tokens = jnp.sum(cntv0)
    num_topk = (num_tokens + 9) // 10

    def pick_digit(remaining):
        def scan_body(i, carry):
            acc, found, digit, greater, bincnt = carry
            h = hist_v[pl.ds(240 - 16 * i, L)]
            rh = lax.rev(h, (0,))
            rcs = plsc.cumsum(rh)
            g_cs = acc + rcs
            ge = g_cs >= remaining
            pc = plsc.all_reduce_population_count(ge)
            hit_cnt = pc[0]
            m_local = 16 - hit_cnt
            loc_digit = 255 - 16 * i - m_local
            loc_cs = jnp.min(jnp.where(ge, g_cs, big))
            loc_greater = jnp.min(jnp.where(ge, g_cs - rh, big))
            loc_binc = loc_cs - loc_greater
            upd = (hit_cnt > 0) & (found == 0)
            digit = jnp.where(upd, loc_digit, digit)
            greater = jnp.where(upd, loc_greater, greater)
            bincnt = jnp.where(upd, loc_binc, bincnt)
            found = found | jnp.where(hit_cnt > 0, 1, 0)
            acc = g_cs[15]
            return acc, found, digit, greater, bincnt

        z = jnp.int32(0)
        _, _, digit, greater, bincnt = lax.fori_loop(
            0, NBVREG, scan_body, (z, z, z, z, z), unroll=4)
        return digit, greater, bincnt

    with jax.named_scope("ph_pick0"):
        digit, greater, bincnt = pick_digit(num_topk)
    prefix = digit
    remaining = num_topk - greater

    for i in range(NBVREG):
        hist_v[pl.ds(i * L, L)] = zeros

    def hist2_body(c, off, prefix=prefix):
        k = keys_v[pl.ds(c * L, L)]
        ub = k ^ MIN32
        byte = lax.shift_right_logical(ub, 16) & 0xFF
        elig = lax.shift_right_logical(ub, 24) == prefix
        plsc.addupdate_scatter(hist_v, [byte], ones, mask=elig)
        plsc.store_compressed(cand1_v.at[pl.ds(off, L)], k, mask=elig)
        pc = plsc.all_reduce_population_count(elig)
        return off + pc[0]

    with jax.named_scope("ph_hist2"):
        @plsc.parallel_loop(0, NCHUNK, unroll=8, carry=jnp.int32(0))
        def n1(c, off):
            return hist2_body(c, off)
    with jax.named_scope("ph_pick"):
        digit, greater, bincnt = pick_digit(remaining)
    prefix = (prefix << 8) | digit
    remaining = remaining - greater

    for i in range(NBVREG):
        hist_v[pl.ds(i * L, L)] = zeros

    def hist1_body(c, off, prefix=prefix):
        base = c * L
        k = cand1_v[pl.ds(base, L)]
        ub = k ^ MIN32
        byte = lax.shift_right_logical(ub, 8) & 0xFF
        valid = (base + iota) < n1
        elig = (lax.shift_right_logical(ub, 16) == prefix) & valid
        plsc.addupdate_scatter(hist_v, [byte], ones, mask=elig)
        plsc.store_compressed(cand2_v.at[pl.ds(off, L)], k, mask=elig)
        pc = plsc.all_reduce_population_count(elig)
        return off + pc[0]

    nv1 = (n1 + L - 1) // L
    with jax.named_scope("ph_hist1"):
        @plsc.parallel_loop(0, nv1, unroll=4, carry=jnp.int32(0))
        def n2(c, off):
            return hist1_body(c, off)
    with jax.named_scope("ph_pick"):
        digit, greater, bincnt = pick_digit(remaining)
    prefix = (prefix << 8) | digit
    remaining = remaining - greater

    for i in range(NBVREG):
        hist_v[pl.ds(i * L, L)] = zeros

    def hist0_body(c, carry, prefix=prefix):
        base = c * L
        k = cand2_v[pl.ds(base, L)]
        ub = k ^ MIN32
        byte = ub & 0xFF
        valid = (base + iota) < n2
        elig = (lax.shift_right_logical(ub, 8) == prefix) & valid
        plsc.addupdate_scatter(hist_v, [byte], ones, mask=elig)
        return carry

    nv2 = (n2 + L - 1) // L
    with jax.named_scope("ph_hist0"):
        lax.fori_loop(0, nv2, hist0_body, 0)
    with jax.named_scope("ph_pick"):
        digit, greater, bincnt = pick_digit(remaining)
    prefix = (prefix << 8) | digit
    remaining = remaining - greater

    t_key = prefix ^ MIN32
    m_eq = remaining
    total_eq = bincnt

    kvec = jnp.broadcast_to(num_topk, (L,)).astype(jnp.float32)
    invv = jnp.full((L,), 1.0, jnp.float32) / kvec
    zerofv = jnp.zeros((L,), jnp.float32)

    @pl.when(m_eq == total_eq)
    def _no_tie():
        def out_body(c, carry):
            k = keys_v[pl.ds(c * L, L)]
            out_v[pl.ds(c * L, L)] = jnp.where(k >= t_key, invv, zerofv)
            return carry

        with jax.named_scope("ph_out"):
            @plsc.parallel_loop(0, NCHUNK, unroll=8)
            def _outl(c):
                out_body(c, 0)

    @pl.when(m_eq != total_eq)
    def _tie():
        def eqcnt_body(c, tot):
            k = keys_v[pl.ds(c * L, L)]
            eq = k == t_key
            pc = plsc.all_reduce_population_count(eq)
            e = pc[0]
            eqc_v[c] = e
            return tot + e

        tot = lax.fori_loop(0, NCHUNK, eqcnt_body, jnp.int32(0), unroll=4)

        def sfx_body(c, run):
            e = eqc_v[c]
            sfx_v[c] = tot - run - e
            return run + e

        lax.fori_loop(0, NCHUNK, sfx_body, jnp.int32(0), unroll=4)

        def out_body(c, carry):
            k = keys_v[pl.ds(c * L, L)]
            eq = k == t_key
            eqi = jnp.where(eq, ones, zeros)
            at_or_after = lax.rev(plsc.cumsum(lax.rev(eqi, (0,))), (0,))
            eq_after = at_or_after - eqi + sfx_v[c]
            sel = (k > t_key) | (eq & (eq_after < m_eq))
            out_v[pl.ds(c * L, L)] = jnp.where(sel, invv, zerofv)
            return carry

        lax.fori_loop(0, NCHUNK, out_body, 0, unroll=4)

    with jax.named_scope("ph_wb"):
        pltpu.sync_copy(out_v, out_hbm.at[b])


def _make_sc_kernel(interpret=False):
    mesh = plsc.VectorSubcoreMesh(core_axis_name="c", subcore_axis_name="s",
                                  num_cores=2, num_subcores=16)

    @functools.partial(
        pl.kernel,
        out_type=jax.ShapeDtypeStruct((B, S), jnp.float32),
        mesh=mesh,
        scratch_types=[
            pltpu.VMEM((H, S), jnp.float32),
            pltpu.VMEM((S,), jnp.int32),
            pltpu.VMEM((NBIN,), jnp.int32),
            pltpu.VMEM((S + L,), jnp.int32),
            pltpu.VMEM((S + L,), jnp.int32),
            pltpu.SMEM((NCHUNK,), jnp.int32),
            pltpu.SMEM((NCHUNK,), jnp.int32),
            pltpu.VMEM((S,), jnp.float32),
            pltpu.VMEM((S,), jnp.float32),
            pltpu.SemaphoreType.DMA,
            pltpu.SemaphoreType.DMA,
        ],
        compiler_params=pltpu.CompilerParams(needs_layout_passes=False),
        interpret=interpret,
    )
    def sc_topk(att_hbm, out_hbm, rows_v, keys_v, hist_v, cand1_v, cand2_v,
                eqc_v, sfx_v, out_v, psum_v, sem, sem2):
        wid = lax.axis_index("s") * 2 + lax.axis_index("c")

        @pl.when(wid < B)
        def _():
            _worker_body(wid, att_hbm, out_hbm, rows_v, keys_v, hist_v,
                         cand1_v, cand2_v, eqc_v, sfx_v, out_v, psum_v,
                         sem, sem2)

    return sc_topk


_sc_topk = _make_sc_kernel()


def kernel(attention):
    return (_sc_topk(attention), None)
